# baseline (speedup 1.0000x reference)
import functools

import jax
import jax.numpy as jnp
from jax import lax
from jax.experimental import pallas as pl
from jax.experimental.pallas import tpu as pltpu
from jax.experimental.pallas import tpu_sc as plsc

_LANES = 16
_NC = 2
_NS = 16
_NW = _NC * _NS
_CH = 80


def _pad_rows(n):
    m = 1024 * _NS
    np_ = ((n + 1023) // 1024) * 1024
    while np_ % (_NS * 8) != 0:
        np_ += 1024
    return np_


def _sc_deg(dst3d, n_pad):
    E = dst3d.shape[0] * dst3d.shape[1] * dst3d.shape[2]
    per_tile = E // _NW
    n_chunks = per_tile // _CH
    assert per_tile * _NW == E and n_chunks * _CH == per_tile
    rows_tile = n_pad // _NS
    mesh = plsc.VectorSubcoreMesh(core_axis_name="c", subcore_axis_name="s")
    vmem = pltpu.VMEM @ mesh
    shared = pltpu.VMEM_SHARED @ mesh
    fire = 5
    assert n_chunks % fire == 0

    @functools.partial(
        pl.kernel,
        mesh=mesh,
        out_type=jax.ShapeDtypeStruct((_NC, n_pad), jnp.float32),
        scratch_types=[
            vmem((n_chunks, _CH), jnp.int32),
            vmem((_CH,), jnp.float32),
            vmem((rows_tile,), jnp.float32),
            shared((n_pad,), jnp.float32),
            pltpu.SemaphoreType.DMA @ mesh,
        ],
    )
    def k(dst_hbm, out_hbm, idx_v, ones_v, zbuf_v, acc_sh, sem):
        c = lax.axis_index("c")
        s = lax.axis_index("s")
        wid = s * _NC + c
        for j in range(_CH // _LANES):
            ones_v[pl.ds(j * _LANES, _LANES)] = jnp.full((_LANES,), 1.0, jnp.float32)

        def zbody(i, carry):
            zbuf_v[pl.ds(i * _LANES, _LANES)] = jnp.zeros((_LANES,), jnp.float32)
            return carry

        lax.fori_loop(0, rows_tile // _LANES, zbody, 0)

        pltpu.sync_copy(dst_hbm.at[wid], idx_v)
        pltpu.sync_copy(zbuf_v, acc_sh.at[pl.ds(s * rows_tile, rows_tile)])
        plsc.subcore_barrier()

        def body(i, carry):
            for b in range(fire):
                pltpu.async_copy(
                    ones_v, acc_sh.at[idx_v.at[i * fire + b]], sem, add=True
                )
            for b in range(fire):
                pltpu.make_async_copy(
                    ones_v, acc_sh.at[idx_v.at[i * fire + b]], sem
                ).wait()
            return carry

        lax.fori_loop(0, n_chunks // fire, body, 0)
        plsc.subcore_barrier()
        pltpu.sync_copy(
            acc_sh.at[pl.ds(s * rows_tile, rows_tile)],
            out_hbm.at[c, pl.ds(s * rows_tile, rows_tile)],
        )

    return k(dst3d)


def _sc_scatter(hs_pad, src2, dst4):
    n_pad, D = hs_pad.shape
    E = src2.shape[0] * src2.shape[1]
    per_tile = E // _NW
    n_chunks = per_tile // _CH
    assert per_tile * _NW == E and n_chunks * _CH == per_tile
    pre = n_chunks % 3
    rows_tile = n_pad // _NS
    mesh = plsc.VectorSubcoreMesh(core_axis_name="c", subcore_axis_name="s")
    vmem = pltpu.VMEM @ mesh
    shared = pltpu.VMEM_SHARED @ mesh
    dma = pltpu.SemaphoreType.DMA @ mesh

    @functools.partial(
        pl.kernel,
        mesh=mesh,
        out_type=jax.ShapeDtypeStruct((_NC, n_pad, D), jnp.float32),
        scratch_types=[
            vmem((per_tile,), jnp.int32),
            vmem((1, _CH), jnp.int32),
            vmem((1, _CH), jnp.int32),
            vmem((1, _CH), jnp.int32),
            vmem((_CH, D), jnp.float32),
            vmem((_CH, D), jnp.float32),
            vmem((_CH, D), jnp.float32),
            dma, dma, dma, dma, dma, dma, dma, dma, dma,
            shared((n_pad, D), jnp.float32),
        ],
    )
    def k(hs_hbm, src_hbm, dst_hbm, out_hbm, sidx_v, di0, di1, di2,
          r0, r1, r2, gs0, gs1, gs2, ds0, ds1, ds2, ss0, ss1, ss2, acc_sh):
        c = lax.axis_index("c")
        s = lax.axis_index("s")
        wid = s * _NC + c
        rows = (r0, r1, r2)
        didx = (di0, di1, di2)
        gsems = (gs0, gs1, gs2)
        dsems = (ds0, ds1, ds2)
        ssems = (ss0, ss1, ss2)
        cbase = wid * n_chunks

        def gath(j, b, sync=False):
            cp = pltpu.async_copy(
                hs_hbm.at[sidx_v.at[pl.ds(j * _CH, _CH)]], rows[b], gsems[b])
            if sync:
                cp.wait()

        def gwait(j, b):
            pltpu.make_async_copy(
                hs_hbm.at[sidx_v.at[pl.ds(j * _CH, _CH)]], rows[b], gsems[b]
            ).wait()

        def dload(j, b):
            pltpu.async_copy(dst_hbm.at[cbase + j], didx[b], dsems[b])

        def dwait(j, b):
            pltpu.make_async_copy(dst_hbm.at[cbase + j], didx[b], dsems[b]).wait()

        def scat(b):
            pltpu.async_copy(rows[b], acc_sh.at[didx[b].at[0]], ssems[b], add=True)

        def swait(b):
            pltpu.make_async_copy(rows[b], acc_sh.at[didx[b].at[0]], ssems[b]).wait()

        pltpu.sync_copy(src_hbm.at[wid], sidx_v)

        @pl.when(c == 0)
        def _():
            pltpu.sync_copy(
                hs_hbm.at[pl.ds(s * rows_tile, rows_tile)],
                acc_sh.at[pl.ds(s * rows_tile, rows_tile)],
            )

        @pl.when(c == 1)
        def _():
            def zbody(i, carry):
                for j in range(D // _LANES):
                    r0[i, pl.ds(j * _LANES, _LANES)] = jnp.zeros(
                        (_LANES,), jnp.float32)
                return carry

            lax.fori_loop(0, _CH, zbody, 0)

            def zcopy(r, carry):
                pltpu.sync_copy(
                    r0, acc_sh.at[pl.ds(s * rows_tile + r * _CH, _CH)]
                )
                return carry

            lax.fori_loop(0, rows_tile // _CH, zcopy, 0)

        plsc.subcore_barrier()

        for j in range(pre):
            dload(j, 0)
            dwait(j, 0)
            gath(j, 0, sync=True)
            scat(0)
            swait(0)

        for b in range(2):
            dload(pre + b, b)
            gath(pre + b, b)

        def body(kk, carry):
            for b in range(3):
                j = pre + 3 * kk + b
                bp = (b + 2) % 3
                dwait(j, b)
                gwait(j, b)
                scat(b)
                if b == 0:
                    @pl.when(kk >= 1)
                    def _():
                        swait(bp)
                else:
                    swait(bp)

                @pl.when(j + 2 < n_chunks)
                def _():
                    dload(j + 2, bp)
                    gath(j + 2, bp)
            return carry

        lax.fori_loop(0, (n_chunks - pre) // 3, body, 0)
        swait((n_chunks - pre - 1) % 3)
        plsc.subcore_barrier()
        pltpu.sync_copy(
            acc_sh.at[pl.ds(s * rows_tile, rows_tile)],
            out_hbm.at[c, pl.ds(s * rows_tile, rows_tile)],
        )

    return k(hs_pad, src2, dst4)


_BR = 1024


def _k1_body(x_ref, w_ref, d0_ref, d1_ref, hs_ref):
    deg = d0_ref[...] + d1_ref[...] + 1.0
    dinv = lax.rsqrt(deg)
    h = jnp.dot(x_ref[...], w_ref[...], preferred_element_type=jnp.float32)
    hs_ref[...] = h * dinv


def _k1(x_pad, Wg, d0, d1):
    n_pad, Din = x_pad.shape
    H = Wg.shape[1]
    grid = (n_pad // _BR,)
    return pl.pallas_call(
        _k1_body,
        grid=grid,
        in_specs=[
            pl.BlockSpec((_BR, Din), lambda i: (i, 0)),
            pl.BlockSpec((Din, H), lambda i: (0, 0)),
            pl.BlockSpec((_BR, 1), lambda i: (i, 0)),
            pl.BlockSpec((_BR, 1), lambda i: (i, 0)),
        ],
        out_specs=pl.BlockSpec((_BR, H), lambda i: (i, 0)),
        out_shape=jax.ShapeDtypeStruct((n_pad, H), jnp.float32),
    )(x_pad, Wg, d0, d1)


def _ln(y, w, b, eps=1e-5):
    mu = jnp.mean(y, axis=-1, keepdims=True)
    var = jnp.mean((y - mu) ** 2, axis=-1, keepdims=True)
    return (y - mu) * lax.rsqrt(var + eps) * w + b


def _k2_body(x_ref, s0_ref, s1_ref, d0_ref, d1_ref, bg_ref,
             w1_ref, b1_ref, w2_ref, b2_ref, l1w_ref, l1b_ref,
             l2w_ref, l2b_ref, o_ref):
    deg = d0_ref[...] + d1_ref[...] + 1.0
    dinv = lax.rsqrt(deg)
    conv = dinv * (s0_ref[0] + s1_ref[0]) + bg_ref[...]
    y = _ln(x_ref[...] + conv, l1w_ref[...], l1b_ref[...])
    t = jnp.maximum(
        jnp.dot(y, w1_ref[...], preferred_element_type=jnp.float32) + b1_ref[...],
        0.0,
    )
    f = jnp.dot(t, w2_ref[...], preferred_element_type=jnp.float32) + b2_ref[...]
    o_ref[...] = _ln(y + f, l2w_ref[...], l2b_ref[...])


def _k2k1_body(x_ref, s0_ref, s1_ref, d0_ref, d1_ref, bg_ref,
               w1_ref, b1_ref, w2_ref, b2_ref, l1w_ref, l1b_ref,
               l2w_ref, l2b_ref, wg_ref, o_ref, hs_ref):
    deg = d0_ref[...] + d1_ref[...] + 1.0
    dinv = lax.rsqrt(deg)
    conv = dinv * (s0_ref[0] + s1_ref[0]) + bg_ref[...]
    y = _ln(x_ref[...] + conv, l1w_ref[...], l1b_ref[...])
    t = jnp.maximum(
        jnp.dot(y, w1_ref[...], preferred_element_type=jnp.float32) + b1_ref[...],
        0.0,
    )
    f = jnp.dot(t, w2_ref[...], preferred_element_type=jnp.float32) + b2_ref[...]
    o = _ln(y + f, l2w_ref[...], l2b_ref[...])
    o_ref[...] = o
    hs_ref[...] = dinv * jnp.dot(
        o, wg_ref[...], preferred_element_type=jnp.float32)


def _k2_specs(n_pad, H, F):
    return [
        pl.BlockSpec((_BR, H), lambda i: (i, 0)),
        pl.BlockSpec((1, _BR, H), lambda i: (0, i, 0)),
        pl.BlockSpec((1, _BR, H), lambda i: (1, i, 0)),
        pl.BlockSpec((_BR, 1), lambda i: (i, 0)),
        pl.BlockSpec((_BR, 1), lambda i: (i, 0)),
        pl.BlockSpec((1, H), lambda i: (0, 0)),
        pl.BlockSpec((H, F), lambda i: (0, 0)),
        pl.BlockSpec((1, F), lambda i: (0, 0)),
        pl.BlockSpec((F, H), lambda i: (0, 0)),
        pl.BlockSpec((1, H), lambda i: (0, 0)),
        pl.BlockSpec((1, H), lambda i: (0, 0)),
        pl.BlockSpec((1, H), lambda i: (0, 0)),
        pl.BlockSpec((1, H), lambda i: (0, 0)),
        pl.BlockSpec((1, H), lambda i: (0, 0)),
    ]


def _row(v):
    return v.reshape(1, -1)


def _k2_args(x_pad, S, d0, d1, p):
    return (x_pad, S, S, d0, d1, _row(p['bg']),
            p['W1'], _row(p['b1']), p['W2'], _row(p['b2']),
            _row(p['ln1_w']), _row(p['ln1_b']),
            _row(p['ln2_w']), _row(p['ln2_b']))


def _k2(x_pad, S, d0, d1, p):
    n_pad, H = x_pad.shape
    F = p['W1'].shape[1]
    return pl.pallas_call(
        _k2_body,
        grid=(n_pad // _BR,),
        in_specs=_k2_specs(n_pad, H, F),
        out_specs=pl.BlockSpec((_BR, H), lambda i: (i, 0)),
        out_shape=jax.ShapeDtypeStruct((n_pad, H), jnp.float32),
    )(*_k2_args(x_pad, S, d0, d1, p))


def _k2k1(x_pad, S, d0, d1, p, wg_next):
    n_pad, H = x_pad.shape
    F = p['W1'].shape[1]
    specs = _k2_specs(n_pad, H, F)
    specs.append(pl.BlockSpec((H, H), lambda i: (0, 0)))
    return pl.pallas_call(
        _k2k1_body,
        grid=(n_pad // _BR,),
        in_specs=specs,
        out_specs=(
            pl.BlockSpec((_BR, H), lambda i: (i, 0)),
            pl.BlockSpec((_BR, H), lambda i: (i, 0)),
        ),
        out_shape=(
            jax.ShapeDtypeStruct((n_pad, H), jnp.float32),
            jax.ShapeDtypeStruct((n_pad, H), jnp.float32),
        ),
    )(*_k2_args(x_pad, S, d0, d1, p), wg_next)


def kernel(x, edge_index, params):
    n, d_in = x.shape
    n_pad = _pad_rows(n)
    src2 = edge_index[0].reshape(_NW, -1)
    dst3d = edge_index[1].reshape(_NW, -1, _CH)
    dst4 = edge_index[1].reshape(-1, 1, _CH)
    x_pad = jnp.pad(x, ((0, n_pad - n), (0, 0)))
    degp = _sc_deg(dst3d, n_pad)
    d0 = degp[0][:, None]
    d1 = degp[1][:, None]
    hs = _k1(x_pad, params[0]['Wg'], d0, d1)
    for i, p in enumerate(params):
        S = _sc_scatter(hs, src2, dst4)
        if i + 1 < len(params):
            x_pad, hs = _k2k1(x_pad, S, d0, d1, p, params[i + 1]['Wg'])
        else:
            x_pad = _k2(x_pad, S, d0, d1, p)
    return x_pad[:n]


# speedup vs baseline: 1.0275x; 1.0275x over previous
import functools

import jax
import jax.numpy as jnp
from jax import lax
from jax.experimental import pallas as pl
from jax.experimental.pallas import tpu as pltpu
from jax.experimental.pallas import tpu_sc as plsc

_LANES = 16
_NC = 2
_NS = 16
_NW = _NC * _NS
_CH = 80


def _acc_rows(n):
    m = _NS * _CH
    return ((n + m - 1) // m) * m


def _sc_deg(ei, n_pad):
    E = ei.shape[0] // 2
    per_tile = E // _NW
    n_chunks = per_tile // _CH
    assert per_tile * _NW == E and n_chunks * _CH == per_tile
    rows_tile = n_pad // _NS
    mesh = plsc.VectorSubcoreMesh(core_axis_name="c", subcore_axis_name="s")
    vmem = pltpu.VMEM @ mesh
    shared = pltpu.VMEM_SHARED @ mesh
    dma = pltpu.SemaphoreType.DMA @ mesh
    fire = 5
    assert n_chunks % fire == 0

    @functools.partial(
        pl.kernel,
        mesh=mesh,
        out_type=jax.ShapeDtypeStruct((_NC, n_pad), jnp.float32),
        scratch_types=[
            [vmem((_CH,), jnp.int32) for _ in range(fire)],
            vmem((_CH,), jnp.float32),
            vmem((rows_tile,), jnp.float32),
            shared((n_pad,), jnp.float32),
            [dma for _ in range(fire)],
            dma,
        ],
    )
    def k(ei_hbm, out_hbm, didx, ones_v, zbuf_v, acc_sh, dsems, ssem):
        c = lax.axis_index("c")
        s = lax.axis_index("s")
        wid = s * _NC + c
        ebase = wid * per_tile
        for j in range(_CH // _LANES):
            ones_v[pl.ds(j * _LANES, _LANES)] = jnp.full((_LANES,), 1.0, jnp.float32)

        def zbody(i, carry):
            zbuf_v[pl.ds(i * _LANES, _LANES)] = jnp.zeros((_LANES,), jnp.float32)
            return carry

        lax.fori_loop(0, rows_tile // _LANES, zbody, 0)
        pltpu.sync_copy(zbuf_v, acc_sh.at[pl.ds(s * rows_tile, rows_tile)])
        plsc.subcore_barrier()

        def body(i, carry):
            for b in range(fire):
                pltpu.async_copy(
                    ei_hbm.at[pl.ds(E + ebase + (i * fire + b) * _CH, _CH)],
                    didx[b], dsems[b])
            for b in range(fire):
                pltpu.make_async_copy(
                    ei_hbm.at[pl.ds(E + ebase + (i * fire + b) * _CH, _CH)],
                    didx[b], dsems[b]).wait()
                pltpu.async_copy(ones_v, acc_sh.at[didx[b]], ssem, add=True)
            for b in range(fire):
                pltpu.make_async_copy(ones_v, acc_sh.at[didx[b]], ssem).wait()
            return carry

        lax.fori_loop(0, n_chunks // fire, body, 0)
        plsc.subcore_barrier()
        pltpu.sync_copy(
            acc_sh.at[pl.ds(s * rows_tile, rows_tile)],
            out_hbm.at[c, pl.ds(s * rows_tile, rows_tile)],
        )

    return k(ei)


def _sc_scatter(hs, ei, n_pad):
    n_real, D = hs.shape
    E = ei.shape[0] // 2
    per_tile = E // _NW
    n_chunks = per_tile // _CH
    assert per_tile * _NW == E and n_chunks * _CH == per_tile
    pre = n_chunks % 3
    rows_tile = n_pad // _NS
    full_tiles = n_real // rows_tile
    tail_rows = n_real - full_tiles * rows_tile
    assert tail_rows % _CH == 0 and rows_tile % _CH == 0
    mesh = plsc.VectorSubcoreMesh(core_axis_name="c", subcore_axis_name="s")
    vmem = pltpu.VMEM @ mesh
    shared = pltpu.VMEM_SHARED @ mesh
    dma = pltpu.SemaphoreType.DMA @ mesh

    @functools.partial(
        pl.kernel,
        mesh=mesh,
        out_type=jax.ShapeDtypeStruct((_NC, n_pad, D), jnp.float32),
        scratch_types=[
            vmem((per_tile,), jnp.int32),
            [vmem((_CH,), jnp.int32) for _ in range(3)],
            [vmem((_CH, D), jnp.float32) for _ in range(3)],
            [dma for _ in range(3)],
            [dma for _ in range(3)],
            [dma for _ in range(3)],
            shared((n_pad, D), jnp.float32),
        ],
    )
    def k(hs_hbm, ei_hbm, out_hbm, sidx_v, didx, rows, gsems, dsems, ssems,
          acc_sh):
        c = lax.axis_index("c")
        s = lax.axis_index("s")
        wid = s * _NC + c
        ebase = wid * per_tile

        def gath(j, b):
            return pltpu.async_copy(
                hs_hbm.at[sidx_v.at[pl.ds(j * _CH, _CH)]], rows[b], gsems[b])

        def gwait(j, b):
            pltpu.make_async_copy(
                hs_hbm.at[sidx_v.at[pl.ds(j * _CH, _CH)]], rows[b], gsems[b]
            ).wait()

        def dload(j, b):
            pltpu.async_copy(
                ei_hbm.at[pl.ds(E + ebase + j * _CH, _CH)], didx[b], dsems[b])

        def dwait(j, b):
            pltpu.make_async_copy(
                ei_hbm.at[pl.ds(E + ebase + j * _CH, _CH)], didx[b], dsems[b]
            ).wait()

        def scat(b):
            pltpu.async_copy(rows[b], acc_sh.at[didx[b]], ssems[b], add=True)

        def swait(b):
            pltpu.make_async_copy(rows[b], acc_sh.at[didx[b]], ssems[b]).wait()

        pltpu.sync_copy(ei_hbm.at[pl.ds(ebase, per_tile)], sidx_v)

        def zbody(i, carry):
            for j in range(D // _LANES):
                rows[0][i, pl.ds(j * _LANES, _LANES)] = jnp.zeros(
                    (_LANES,), jnp.float32)
            return carry

        lax.fori_loop(0, _CH, zbody, 0)

        def zfill(base_row, n_rows):
            def zcopy(r, carry):
                pltpu.sync_copy(
                    rows[0], acc_sh.at[pl.ds(base_row + r * _CH, _CH)])
                return carry
            lax.fori_loop(0, n_rows // _CH, zcopy, 0)

        @pl.when(c == 0)
        def _():
            @pl.when(s < full_tiles)
            def _():
                pltpu.sync_copy(
                    hs_hbm.at[pl.ds(s * rows_tile, rows_tile)],
                    acc_sh.at[pl.ds(s * rows_tile, rows_tile)],
                )

            @pl.when(s == full_tiles)
            def _():
                if tail_rows:
                    pltpu.sync_copy(
                        hs_hbm.at[pl.ds(full_tiles * rows_tile, tail_rows)],
                        acc_sh.at[pl.ds(full_tiles * rows_tile, tail_rows)],
                    )
                zfill(full_tiles * rows_tile + tail_rows,
                      rows_tile - tail_rows)

            @pl.when(s > full_tiles)
            def _():
                zfill(s * rows_tile, rows_tile)

        @pl.when(c == 1)
        def _():
            zfill(s * rows_tile, rows_tile)

        plsc.subcore_barrier()

        for j in range(pre):
            dload(j, 0)
            dwait(j, 0)
            gath(j, 0).wait()
            scat(0)
            swait(0)

        for b in range(2):
            dload(pre + b, b)
            gath(pre + b, b)

        def body(kk, carry):
            for b in range(3):
                j = pre + 3 * kk + b
                bp = (b + 2) % 3
                dwait(j, b)
                gwait(j, b)
                scat(b)
                if b == 0:
                    @pl.when(kk >= 1)
                    def _():
                        swait(bp)
                else:
                    swait(bp)

                @pl.when(j + 2 < n_chunks)
                def _():
                    dload(j + 2, bp)
                    gath(j + 2, bp)
            return carry

        lax.fori_loop(0, (n_chunks - pre) // 3, body, 0)
        swait((n_chunks - pre - 1) % 3)
        plsc.subcore_barrier()
        pltpu.sync_copy(
            acc_sh.at[pl.ds(s * rows_tile, rows_tile)],
            out_hbm.at[c, pl.ds(s * rows_tile, rows_tile)],
        )

    return k(hs, ei)


_BR = 1000


def _k1_body(x_ref, w_ref, d0_ref, d1_ref, hs_ref):
    deg = d0_ref[...] + d1_ref[...] + 1.0
    dinv = lax.rsqrt(deg)
    h = jnp.dot(x_ref[...], w_ref[...], preferred_element_type=jnp.float32)
    hs_ref[...] = h * dinv


def _k1(x, Wg, d0, d1):
    n, Din = x.shape
    H = Wg.shape[1]
    return pl.pallas_call(
        _k1_body,
        grid=(n // _BR,),
        in_specs=[
            pl.BlockSpec((_BR, Din), lambda i: (i, 0)),
            pl.BlockSpec((Din, H), lambda i: (0, 0)),
            pl.BlockSpec((_BR, 1), lambda i: (i, 0)),
            pl.BlockSpec((_BR, 1), lambda i: (i, 0)),
        ],
        out_specs=pl.BlockSpec((_BR, H), lambda i: (i, 0)),
        out_shape=jax.ShapeDtypeStruct((n, H), jnp.float32),
    )(x, Wg, d0, d1)


def _ln(y, w, b, eps=1e-5):
    mu = jnp.mean(y, axis=-1, keepdims=True)
    var = jnp.mean((y - mu) ** 2, axis=-1, keepdims=True)
    return (y - mu) * lax.rsqrt(var + eps) * w + b


def _k2_core(x_ref, s0_ref, s1_ref, d0_ref, d1_ref, bg_ref,
             w1_ref, b1_ref, w2_ref, b2_ref, l1w_ref, l1b_ref,
             l2w_ref, l2b_ref):
    deg = d0_ref[...] + d1_ref[...] + 1.0
    dinv = lax.rsqrt(deg)
    conv = dinv * (s0_ref[0] + s1_ref[0]) + bg_ref[...]
    y = _ln(x_ref[...] + conv, l1w_ref[...], l1b_ref[...])
    t = jnp.maximum(
        jnp.dot(y, w1_ref[...], preferred_element_type=jnp.float32) + b1_ref[...],
        0.0,
    )
    f = jnp.dot(t, w2_ref[...], preferred_element_type=jnp.float32) + b2_ref[...]
    return _ln(y + f, l2w_ref[...], l2b_ref[...]), dinv


def _k2_body(*refs):
    *ins, o_ref = refs
    o, _ = _k2_core(*ins)
    o_ref[...] = o


def _k2k1_body(*refs):
    *ins, wg_ref, o_ref, hs_ref = refs
    o, dinv = _k2_core(*ins)
    o_ref[...] = o
    hs_ref[...] = dinv * jnp.dot(
        o, wg_ref[...], preferred_element_type=jnp.float32)


def _k2_specs(H, F):
    return [
        pl.BlockSpec((_BR, H), lambda i: (i, 0)),
        pl.BlockSpec((1, _BR, H), lambda i: (0, i, 0)),
        pl.BlockSpec((1, _BR, H), lambda i: (1, i, 0)),
        pl.BlockSpec((_BR, 1), lambda i: (i, 0)),
        pl.BlockSpec((_BR, 1), lambda i: (i, 0)),
        pl.BlockSpec((1, H), lambda i: (0, 0)),
        pl.BlockSpec((H, F), lambda i: (0, 0)),
        pl.BlockSpec((1, F), lambda i: (0, 0)),
        pl.BlockSpec((F, H), lambda i: (0, 0)),
        pl.BlockSpec((1, H), lambda i: (0, 0)),
        pl.BlockSpec((1, H), lambda i: (0, 0)),
        pl.BlockSpec((1, H), lambda i: (0, 0)),
        pl.BlockSpec((1, H), lambda i: (0, 0)),
        pl.BlockSpec((1, H), lambda i: (0, 0)),
    ]


def _row(v):
    return v.reshape(1, -1)


def _k2_args(x, S, d0, d1, p):
    return (x, S, S, d0, d1, _row(p['bg']),
            p['W1'], _row(p['b1']), p['W2'], _row(p['b2']),
            _row(p['ln1_w']), _row(p['ln1_b']),
            _row(p['ln2_w']), _row(p['ln2_b']))


def _k2(x, S, d0, d1, p):
    n, H = x.shape
    F = p['W1'].shape[1]
    return pl.pallas_call(
        _k2_body,
        grid=(n // _BR,),
        in_specs=_k2_specs(H, F),
        out_specs=pl.BlockSpec((_BR, H), lambda i: (i, 0)),
        out_shape=jax.ShapeDtypeStruct((n, H), jnp.float32),
    )(*_k2_args(x, S, d0, d1, p))


def _k2k1(x, S, d0, d1, p, wg_next):
    n, H = x.shape
    F = p['W1'].shape[1]
    specs = _k2_specs(H, F)
    specs.append(pl.BlockSpec((H, H), lambda i: (0, 0)))
    return pl.pallas_call(
        _k2k1_body,
        grid=(n // _BR,),
        in_specs=specs,
        out_specs=(
            pl.BlockSpec((_BR, H), lambda i: (i, 0)),
            pl.BlockSpec((_BR, H), lambda i: (i, 0)),
        ),
        out_shape=(
            jax.ShapeDtypeStruct((n, H), jnp.float32),
            jax.ShapeDtypeStruct((n, H), jnp.float32),
        ),
    )(*_k2_args(x, S, d0, d1, p), wg_next)


def kernel(x, edge_index, params):
    n, d_in = x.shape
    n_pad = _acc_rows(n)
    ei = edge_index.reshape(-1)
    degp = _sc_deg(ei, n_pad)
    d0 = degp[0, :n][:, None]
    d1 = degp[1, :n][:, None]
    hs = _k1(x, params[0]['Wg'], d0, d1)
    for i, p in enumerate(params):
        S = _sc_scatter(hs, ei, n_pad)
        if i + 1 < len(params):
            x, hs = _k2k1(x, S, d0, d1, p, params[i + 1]['Wg'])
        else:
            x = _k2(x, S, d0, d1, p)
    return x


# speedup vs baseline: 1.0668x; 1.0382x over previous
import functools

import jax
import jax.numpy as jnp
from jax import lax
from jax.experimental import pallas as pl
from jax.experimental.pallas import tpu as pltpu
from jax.experimental.pallas import tpu_sc as plsc

_LANES = 16
_NC = 2
_NS = 16
_NW = _NC * _NS
_CH = 80


def _acc_rows(n):
    m = _NS * _CH
    return ((n + m - 1) // m) * m


def _sc_deg(ei, n_pad):
    E = ei.shape[0] // 2
    per_tile = E // _NW
    n_chunks = per_tile // _CH
    assert per_tile * _NW == E and n_chunks * _CH == per_tile
    rows_tile = n_pad // _NS
    mesh = plsc.VectorSubcoreMesh(core_axis_name="c", subcore_axis_name="s")
    vmem = pltpu.VMEM @ mesh
    shared = pltpu.VMEM_SHARED @ mesh
    dma = pltpu.SemaphoreType.DMA @ mesh
    fire = 5
    assert n_chunks % fire == 0

    @functools.partial(
        pl.kernel,
        mesh=mesh,
        out_type=jax.ShapeDtypeStruct((_NC, n_pad), jnp.float32),
        scratch_types=[
            vmem((per_tile,), jnp.int32),
            vmem((_CH,), jnp.float32),
            vmem((rows_tile,), jnp.float32),
            shared((n_pad,), jnp.float32),
            dma,
        ],
    )
    def k(ei_hbm, out_hbm, didx_v, ones_v, zbuf_v, acc_sh, ssem):
        c = lax.axis_index("c")
        s = lax.axis_index("s")
        wid = s * _NC + c
        ebase = wid * per_tile
        for j in range(_CH // _LANES):
            ones_v[pl.ds(j * _LANES, _LANES)] = jnp.full((_LANES,), 1.0, jnp.float32)

        def zbody(i, carry):
            zbuf_v[pl.ds(i * _LANES, _LANES)] = jnp.zeros((_LANES,), jnp.float32)
            return carry

        lax.fori_loop(0, rows_tile // _LANES, zbody, 0)
        pltpu.sync_copy(ei_hbm.at[pl.ds(E + ebase, per_tile)], didx_v)
        pltpu.sync_copy(zbuf_v, acc_sh.at[pl.ds(s * rows_tile, rows_tile)])
        plsc.subcore_barrier()

        def body(i, carry):
            for b in range(fire):
                pltpu.async_copy(
                    ones_v,
                    acc_sh.at[didx_v.at[pl.ds((i * fire + b) * _CH, _CH)]],
                    ssem, add=True)
            for b in range(fire):
                pltpu.make_async_copy(
                    ones_v,
                    acc_sh.at[didx_v.at[pl.ds((i * fire + b) * _CH, _CH)]],
                    ssem).wait()
            return carry

        lax.fori_loop(0, n_chunks // fire, body, 0)
        plsc.subcore_barrier()
        pltpu.sync_copy(
            acc_sh.at[pl.ds(s * rows_tile, rows_tile)],
            out_hbm.at[c, pl.ds(s * rows_tile, rows_tile)],
        )

    return k(ei)


def _sc_scatter(hs, ei, n_pad):
    n_real, D = hs.shape
    E = ei.shape[0] // 2
    per_tile = E // _NW
    n_chunks = per_tile // _CH
    assert per_tile * _NW == E and n_chunks * _CH == per_tile
    pre = n_chunks % 3
    rows_tile = n_pad // _NS
    full_tiles = n_real // rows_tile
    tail_rows = n_real - full_tiles * rows_tile
    assert tail_rows % _CH == 0 and rows_tile % _CH == 0
    mesh = plsc.VectorSubcoreMesh(core_axis_name="c", subcore_axis_name="s")
    vmem = pltpu.VMEM @ mesh
    shared = pltpu.VMEM_SHARED @ mesh
    dma = pltpu.SemaphoreType.DMA @ mesh

    @functools.partial(
        pl.kernel,
        mesh=mesh,
        out_type=jax.ShapeDtypeStruct((_NC, n_pad, D), jnp.float32),
        scratch_types=[
            vmem((per_tile,), jnp.int32),
            [vmem((_CH,), jnp.int32) for _ in range(3)],
            [vmem((_CH, D), jnp.float32) for _ in range(3)],
            [dma for _ in range(3)],
            [dma for _ in range(3)],
            [dma for _ in range(3)],
            shared((n_pad, D), jnp.float32),
        ],
    )
    def k(hs_hbm, ei_hbm, out_hbm, sidx_v, didx, rows, gsems, dsems, ssems,
          acc_sh):
        c = lax.axis_index("c")
        s = lax.axis_index("s")
        wid = s * _NC + c
        ebase = wid * per_tile

        def gath(j, b):
            return pltpu.async_copy(
                hs_hbm.at[sidx_v.at[pl.ds(j * _CH, _CH)]], rows[b], gsems[b])

        def gwait(j, b):
            pltpu.make_async_copy(
                hs_hbm.at[sidx_v.at[pl.ds(j * _CH, _CH)]], rows[b], gsems[b]
            ).wait()

        def dload(j, b):
            pltpu.async_copy(
                ei_hbm.at[pl.ds(E + ebase + j * _CH, _CH)], didx[b], dsems[b])

        def dwait(j, b):
            pltpu.make_async_copy(
                ei_hbm.at[pl.ds(E + ebase + j * _CH, _CH)], didx[b], dsems[b]
            ).wait()

        def scat(b):
            pltpu.async_copy(rows[b], acc_sh.at[didx[b]], ssems[b], add=True)

        def swait(b):
            pltpu.make_async_copy(rows[b], acc_sh.at[didx[b]], ssems[b]).wait()

        pltpu.sync_copy(ei_hbm.at[pl.ds(ebase, per_tile)], sidx_v)

        def zbody(i, carry):
            for j in range(D // _LANES):
                rows[0][i, pl.ds(j * _LANES, _LANES)] = jnp.zeros(
                    (_LANES,), jnp.float32)
            return carry

        lax.fori_loop(0, _CH, zbody, 0)

        def zfill(base_row, n_rows):
            def zcopy(r, carry):
                pltpu.sync_copy(
                    rows[0], acc_sh.at[pl.ds(base_row + r * _CH, _CH)])
                return carry
            lax.fori_loop(0, n_rows // _CH, zcopy, 0)

        @pl.when(c == 0)
        def _():
            @pl.when(s < full_tiles)
            def _():
                pltpu.sync_copy(
                    hs_hbm.at[pl.ds(s * rows_tile, rows_tile)],
                    acc_sh.at[pl.ds(s * rows_tile, rows_tile)],
                )

            @pl.when(s == full_tiles)
            def _():
                if tail_rows:
                    pltpu.sync_copy(
                        hs_hbm.at[pl.ds(full_tiles * rows_tile, tail_rows)],
                        acc_sh.at[pl.ds(full_tiles * rows_tile, tail_rows)],
                    )
                zfill(full_tiles * rows_tile + tail_rows,
                      rows_tile - tail_rows)

            @pl.when(s > full_tiles)
            def _():
                zfill(s * rows_tile, rows_tile)

        @pl.when(c == 1)
        def _():
            zfill(s * rows_tile, rows_tile)

        plsc.subcore_barrier()

        for j in range(pre):
            dload(j, 0)
            dwait(j, 0)
            gath(j, 0).wait()
            scat(0)
            swait(0)

        for b in range(2):
            dload(pre + b, b)
            gath(pre + b, b)

        def body(kk, carry):
            for b in range(3):
                j = pre + 3 * kk + b
                bp = (b + 2) % 3
                dwait(j, b)
                gwait(j, b)
                scat(b)
                if b == 0:
                    @pl.when(kk >= 1)
                    def _():
                        swait(bp)
                else:
                    swait(bp)

                @pl.when(j + 2 < n_chunks)
                def _():
                    dload(j + 2, bp)
                    gath(j + 2, bp)
            return carry

        lax.fori_loop(0, (n_chunks - pre) // 3, body, 0)
        swait((n_chunks - pre - 1) % 3)
        plsc.subcore_barrier()
        pltpu.sync_copy(
            acc_sh.at[pl.ds(s * rows_tile, rows_tile)],
            out_hbm.at[c, pl.ds(s * rows_tile, rows_tile)],
        )

    return k(hs, ei)


_BR = 1000


def _k1_body(x_ref, w_ref, d0_ref, d1_ref, hs_ref):
    deg = d0_ref[...] + d1_ref[...] + 1.0
    dinv = lax.rsqrt(deg)
    h = jnp.dot(x_ref[...], w_ref[...], preferred_element_type=jnp.float32)
    hs_ref[...] = h * dinv


def _k1(x, Wg, d0, d1):
    n, Din = x.shape
    H = Wg.shape[1]
    return pl.pallas_call(
        _k1_body,
        grid=(n // _BR,),
        in_specs=[
            pl.BlockSpec((_BR, Din), lambda i: (i, 0)),
            pl.BlockSpec((Din, H), lambda i: (0, 0)),
            pl.BlockSpec((_BR, 1), lambda i: (i, 0)),
            pl.BlockSpec((_BR, 1), lambda i: (i, 0)),
        ],
        out_specs=pl.BlockSpec((_BR, H), lambda i: (i, 0)),
        out_shape=jax.ShapeDtypeStruct((n, H), jnp.float32),
    )(x, Wg, d0, d1)


def _ln(y, w, b, eps=1e-5):
    mu = jnp.mean(y, axis=-1, keepdims=True)
    var = jnp.mean((y - mu) ** 2, axis=-1, keepdims=True)
    return (y - mu) * lax.rsqrt(var + eps) * w + b


def _k2_core(x_ref, s0_ref, s1_ref, d0_ref, d1_ref, bg_ref,
             w1_ref, b1_ref, w2_ref, b2_ref, l1w_ref, l1b_ref,
             l2w_ref, l2b_ref):
    deg = d0_ref[...] + d1_ref[...] + 1.0
    dinv = lax.rsqrt(deg)
    conv = dinv * (s0_ref[0] + s1_ref[0]) + bg_ref[...]
    y = _ln(x_ref[...] + conv, l1w_ref[...], l1b_ref[...])
    t = jnp.maximum(
        jnp.dot(y, w1_ref[...], preferred_element_type=jnp.float32) + b1_ref[...],
        0.0,
    )
    f = jnp.dot(t, w2_ref[...], preferred_element_type=jnp.float32) + b2_ref[...]
    return _ln(y + f, l2w_ref[...], l2b_ref[...]), dinv


def _k2_body(*refs):
    *ins, o_ref = refs
    o, _ = _k2_core(*ins)
    o_ref[...] = o


def _k2k1_body(*refs):
    *ins, wg_ref, o_ref, hs_ref = refs
    o, dinv = _k2_core(*ins)
    o_ref[...] = o
    hs_ref[...] = dinv * jnp.dot(
        o, wg_ref[...], preferred_element_type=jnp.float32)


def _k2_specs(H, F):
    return [
        pl.BlockSpec((_BR, H), lambda i: (i, 0)),
        pl.BlockSpec((1, _BR, H), lambda i: (0, i, 0)),
        pl.BlockSpec((1, _BR, H), lambda i: (1, i, 0)),
        pl.BlockSpec((_BR, 1), lambda i: (i, 0)),
        pl.BlockSpec((_BR, 1), lambda i: (i, 0)),
        pl.BlockSpec((1, H), lambda i: (0, 0)),
        pl.BlockSpec((H, F), lambda i: (0, 0)),
        pl.BlockSpec((1, F), lambda i: (0, 0)),
        pl.BlockSpec((F, H), lambda i: (0, 0)),
        pl.BlockSpec((1, H), lambda i: (0, 0)),
        pl.BlockSpec((1, H), lambda i: (0, 0)),
        pl.BlockSpec((1, H), lambda i: (0, 0)),
        pl.BlockSpec((1, H), lambda i: (0, 0)),
        pl.BlockSpec((1, H), lambda i: (0, 0)),
    ]


def _row(v):
    return v.reshape(1, -1)


def _k2_args(x, S, d0, d1, p):
    return (x, S, S, d0, d1, _row(p['bg']),
            p['W1'], _row(p['b1']), p['W2'], _row(p['b2']),
            _row(p['ln1_w']), _row(p['ln1_b']),
            _row(p['ln2_w']), _row(p['ln2_b']))


def _k2(x, S, d0, d1, p):
    n, H = x.shape
    F = p['W1'].shape[1]
    return pl.pallas_call(
        _k2_body,
        grid=(n // _BR,),
        in_specs=_k2_specs(H, F),
        out_specs=pl.BlockSpec((_BR, H), lambda i: (i, 0)),
        out_shape=jax.ShapeDtypeStruct((n, H), jnp.float32),
    )(*_k2_args(x, S, d0, d1, p))


def _k2k1(x, S, d0, d1, p, wg_next):
    n, H = x.shape
    F = p['W1'].shape[1]
    specs = _k2_specs(H, F)
    specs.append(pl.BlockSpec((H, H), lambda i: (0, 0)))
    return pl.pallas_call(
        _k2k1_body,
        grid=(n // _BR,),
        in_specs=specs,
        out_specs=(
            pl.BlockSpec((_BR, H), lambda i: (i, 0)),
            pl.BlockSpec((_BR, H), lambda i: (i, 0)),
        ),
        out_shape=(
            jax.ShapeDtypeStruct((n, H), jnp.float32),
            jax.ShapeDtypeStruct((n, H), jnp.float32),
        ),
    )(*_k2_args(x, S, d0, d1, p), wg_next)


def kernel(x, edge_index, params):
    n, d_in = x.shape
    n_pad = _acc_rows(n)
    ei = edge_index.reshape(-1)
    degp = _sc_deg(ei, n_pad)
    d0 = degp[0, :n][:, None]
    d1 = degp[1, :n][:, None]
    hs = _k1(x, params[0]['Wg'], d0, d1)
    for i, p in enumerate(params):
        S = _sc_scatter(hs, ei, n_pad)
        if i + 1 < len(params):
            x, hs = _k2k1(x, S, d0, d1, p, params[i + 1]['Wg'])
        else:
            x = _k2(x, S, d0, d1, p)
    return x
